# baseline (speedup 1.0000x reference)
import functools

import jax
import jax.numpy as jnp
from jax import lax
from jax.experimental import pallas as pl
from jax.experimental.pallas import tpu as pltpu
from jax.experimental.pallas import tpu_sc as plsc

_M = 8192
_D = 768
_NC = 2
_NS = 16
_NW = _NC * _NS
_ROWS_PER_W = _M // _NW
_CHUNK = 64


def _make_sc_copy():
    mesh = plsc.VectorSubcoreMesh(core_axis_name="c", subcore_axis_name="s")
    n = _ROWS_PER_W // _CHUNK

    @functools.partial(
        pl.kernel,
        mesh=mesh,
        out_type=jax.ShapeDtypeStruct((_M, _D), jnp.float32),
        scratch_types=[
            pltpu.VMEM((2, _CHUNK, _D), jnp.float32),
            pltpu.SemaphoreType.DMA((2,)),
            pltpu.SemaphoreType.DMA((2,)),
        ],
    )
    def sc_copy(src_hbm, out_hbm, bufs, rsem, wsem):
        wid = lax.axis_index("s") * _NC + lax.axis_index("c")
        base = wid * _ROWS_PER_W

        def rcopy(j, b):
            return pltpu.make_async_copy(
                src_hbm.at[pl.ds(base + j * _CHUNK, _CHUNK), :],
                bufs.at[b], rsem.at[b])

        def wcopy(j, b):
            return pltpu.make_async_copy(
                bufs.at[b],
                out_hbm.at[pl.ds(base + j * _CHUNK, _CHUNK), :], wsem.at[b])

        rcopy(0, 0).start()
        for j in range(n):
            b = j % 2
            rcopy(j, b).wait()
            if j > 0:
                wcopy(j - 1, 1 - b).wait()
            wcopy(j, b).start()
            if j + 1 < n:
                rcopy(j + 1, 1 - b).start()
        wcopy(n - 1, (n - 1) % 2).wait()

    return sc_copy


_sc_copy = _make_sc_copy()


def kernel(x, pos_emb):
    del x
    return _sc_copy(pos_emb)


# speedup vs baseline: 2.4223x; 2.4223x over previous
import jax
import jax.numpy as jnp
from jax.experimental import pallas as pl

_BLK = 4096


def _copy_block(src_ref, dst_ref):
    dst_ref[...] = src_ref[...]


def kernel(x, pos_emb):
    del x
    m, d = pos_emb.shape
    return pl.pallas_call(
        _copy_block,
        grid=(m // _BLK,),
        in_specs=[pl.BlockSpec((_BLK, d), lambda i: (i, 0))],
        out_specs=pl.BlockSpec((_BLK, d), lambda i: (i, 0)),
        out_shape=jax.ShapeDtypeStruct((m, d), pos_emb.dtype),
    )(pos_emb)
